# baseline (speedup 1.0000x reference)
import jax
import jax.numpy as jnp
from jax import lax
from jax.experimental import pallas as pl
from jax.experimental.pallas import tpu as pltpu
from jax.experimental.pallas import tpu_sc as plsc

_BINS = (64, 256, 64, 256, 512, 256, 512, 512, 256)
_D = 8
_B = 16384
_F = len(_BINS)
_ROWS = sum(_BINS)
_BASE = tuple(sum(_BINS[:f]) for f in range(_F))
_OC = 128

_NC = 2
_NS = 16
_NW = _NC * _NS
_C = _B // _NW
_G = 16
_NG = _C // _G


def _body(tab, i0, i1, i2, i3, i4, i5, i6, i7, i8,
          out, tab_v, idx_v, oblk, sem):
    idx_refs = (i0, i1, i2, i3, i4, i5, i6, i7, i8)
    cid = lax.axis_index("c")
    sid = lax.axis_index("s")
    wid = sid * _NC + cid
    base = wid * _C

    copies = [(tab, tab_v)]
    for f in range(_F):
        copies.append((idx_refs[f].at[pl.ds(base, _C)],
                       idx_v.at[pl.ds(f * _C, _C)]))
    for src, dst in copies:
        pltpu.async_copy(src, dst, sem)
    for src, dst in copies:
        pltpu.make_async_copy(src, dst, sem).wait()

    lanes = lax.iota(jnp.int32, 16)
    kvecs = tuple(jnp.full((16,), k, jnp.int32) for k in range(_D))

    nch = 4
    gpc = _NG // nch
    rpc = gpc * _G
    wcopies = []
    for c in range(nch):
        for f in range(_F):
            @plsc.parallel_loop(c * gpc, (c + 1) * gpc, unroll=8)
            def group(g, f=f):
                rows = g * _G + lanes
                raw = idx_v[pl.ds(f * _C + g * _G, _G)]
                h = (raw & (_BINS[f] - 1)) + _BASE[f]
                for k in range(_D):
                    val = plsc.load_gather(tab_v, [h + (k * _ROWS)])
                    plsc.store_scatter(oblk, [rows, kvecs[k] + (f * _D)], val)
        wcopies.append((oblk.at[pl.ds(c * rpc, rpc), pl.ds(0, _OC)],
                        out.at[pl.ds(base + c * rpc, rpc)]))
        pltpu.async_copy(*wcopies[-1], sem)
    for src, dst in wcopies:
        pltpu.make_async_copy(src, dst, sem).wait()


@jax.jit
def kernel(idx_0, idx_1, idx_2, idx_3, idx_4, idx_5, idx_6, idx_7, idx_8,
           W_0, W_1, W_2, W_3, W_4, W_5, W_6, W_7, W_8):
    tab = jnp.concatenate(
        [W_0, W_1, W_2, W_3, W_4, W_5, W_6, W_7, W_8], axis=0
    ).T.reshape(_ROWS * _D)
    mesh = plsc.VectorSubcoreMesh(
        core_axis_name="c", subcore_axis_name="s", num_cores=_NC, num_subcores=_NS
    )
    run = pl.kernel(
        _body,
        out_type=jax.ShapeDtypeStruct((_B, _OC), jnp.float32),
        mesh=mesh,
        scratch_types=[
            pltpu.VMEM((_ROWS * _D,), jnp.float32),
            pltpu.VMEM((_F * _C,), jnp.int32),
            pltpu.VMEM((_C, _OC + 1), jnp.float32),
            pltpu.SemaphoreType.DMA,
        ],
        compiler_params=pltpu.CompilerParams(
            needs_layout_passes=False, use_tc_tiling_on_sc=False
        ),
    )
    padded = run(tab, idx_0, idx_1, idx_2, idx_3, idx_4, idx_5, idx_6,
                 idx_7, idx_8)
    return padded[:, : _F * _D]


# speedup vs baseline: 1.1533x; 1.1533x over previous
import jax
import jax.numpy as jnp
from jax import lax
from jax.experimental import pallas as pl
from jax.experimental.pallas import tpu as pltpu
from jax.experimental.pallas import tpu_sc as plsc

_BINS = (64, 256, 64, 256, 512, 256, 512, 512, 256)
_D = 8
_B = 16384
_F = len(_BINS)
_ROWS = sum(_BINS)
_BASE = tuple(sum(_BINS[:f]) for f in range(_F))
_OC = 128

_NC = 2
_NS = 16
_NW = _NC * _NS
_C = _B // _NW
_G = 16
_NG = _C // _G


def _body(tab, i0, i1, i2, i3, i4, i5, i6, i7, i8,
          out, tab_v, idx_v, oblk, sem):
    idx_refs = (i0, i1, i2, i3, i4, i5, i6, i7, i8)
    cid = lax.axis_index("c")
    sid = lax.axis_index("s")
    wid = sid * _NC + cid
    base = wid * _C

    copies = [(tab, tab_v)]
    for f in range(_F):
        copies.append((idx_refs[f].at[pl.ds(base, _C)],
                       idx_v.at[pl.ds(f * _C, _C)]))
    for src, dst in copies:
        pltpu.async_copy(src, dst, sem)
    for src, dst in copies:
        pltpu.make_async_copy(src, dst, sem).wait()

    lanes = lax.iota(jnp.int32, 16)
    kvecs = tuple(jnp.full((16,), k, jnp.int32) for k in range(_D))

    for f in range(_F):
        @plsc.parallel_loop(0, _NG, unroll=16)
        def group(g, f=f):
            rows = g * _G + lanes
            raw = idx_v[pl.ds(f * _C + g * _G, _G)]
            h = (raw & (_BINS[f] - 1)) + _BASE[f]
            for k in range(_D):
                val = plsc.load_gather(tab_v, [h + (k * _ROWS)])
                plsc.store_scatter(oblk, [rows, kvecs[k] + (f * _D)], val)

    pltpu.sync_copy(oblk.at[:, pl.ds(0, _OC)], out.at[pl.ds(base, _C)])


@jax.jit
def kernel(idx_0, idx_1, idx_2, idx_3, idx_4, idx_5, idx_6, idx_7, idx_8,
           W_0, W_1, W_2, W_3, W_4, W_5, W_6, W_7, W_8):
    tab = jnp.concatenate(
        [W_0, W_1, W_2, W_3, W_4, W_5, W_6, W_7, W_8], axis=0
    ).T.reshape(_ROWS * _D)
    mesh = plsc.VectorSubcoreMesh(
        core_axis_name="c", subcore_axis_name="s", num_cores=_NC, num_subcores=_NS
    )
    run = pl.kernel(
        _body,
        out_type=jax.ShapeDtypeStruct((_B, _OC), jnp.float32),
        mesh=mesh,
        scratch_types=[
            pltpu.VMEM((_ROWS * _D,), jnp.float32),
            pltpu.VMEM((_F * _C,), jnp.int32),
            pltpu.VMEM((_C, _OC + 1), jnp.float32),
            pltpu.SemaphoreType.DMA,
        ],
        compiler_params=pltpu.CompilerParams(
            needs_layout_passes=False, use_tc_tiling_on_sc=False
        ),
    )
    padded = run(tab, idx_0, idx_1, idx_2, idx_3, idx_4, idx_5, idx_6,
                 idx_7, idx_8)
    return padded[:, : _F * _D]
